# baseline (speedup 1.0000x reference)
import jax
import jax.numpy as jnp
from jax import lax
from jax.experimental import pallas as pl
from jax.experimental.pallas import tpu as pltpu
from jax.experimental.pallas import tpu_sc as plsc

_NC = 2
_NS = 16
_NW = _NC * _NS
_NBUF = 8
_LANES = 128


def kernel(segment_ids, table):
    batch, hist = segment_ids.shape
    num_rows, dim = table.shape
    rows_per_w = batch // _NW
    hist_pad = 56
    assert batch % _NW == 0 and rows_per_w % _NBUF == 0

    ipad = jnp.pad(segment_ids.astype(jnp.int32), ((0, 0), (0, _LANES - hist)))
    mesh = plsc.VectorSubcoreMesh(core_axis_name="c", subcore_axis_name="s")

    @pl.kernel(
        out_type=jax.ShapeDtypeStruct((batch, hist, dim), table.dtype),
        mesh=mesh,
        scratch_types=[
            pltpu.VMEM((rows_per_w, _LANES), jnp.int32),
            pltpu.VMEM((_NBUF, hist_pad, dim), jnp.float32),
            pltpu.SemaphoreType.DMA((_NBUF,)),
            pltpu.SemaphoreType.DMA((_NBUF,)),
        ],
        compiler_params=pltpu.CompilerParams(use_tc_tiling_on_sc=False),
    )
    def gather_kernel(table_hbm, idx_hbm, out_hbm, idx_v, rows_v, gsem, wsem):
        wid = lax.axis_index("s") * _NC + lax.axis_index("c")
        base = wid * rows_per_w
        pltpu.sync_copy(idx_hbm.at[pl.ds(base, rows_per_w)], idx_v)

        def fire_gather(r, b):
            pltpu.async_copy(
                table_hbm.at[idx_v.at[r, pl.ds(0, hist_pad)]],
                rows_v.at[b],
                gsem.at[b],
            )

        for b in range(_NBUF):
            fire_gather(b, b)

        @pl.loop(0, rows_per_w, step=_NBUF)
        def _(r0):
            for b in range(_NBUF):
                r = r0 + b
                pltpu.make_async_copy(
                    table_hbm.at[idx_v.at[r, pl.ds(0, hist_pad)]],
                    rows_v.at[b],
                    gsem.at[b],
                ).wait()
                wb = pltpu.async_copy(
                    rows_v.at[b, pl.ds(0, hist)],
                    out_hbm.at[base + r],
                    wsem.at[b],
                )

                @pl.when(r + _NBUF < rows_per_w)
                def _():
                    wb.wait()
                    fire_gather(r + _NBUF, b)

        for b in range(_NBUF):
            pltpu.make_async_copy(
                rows_v.at[b, pl.ds(0, hist)],
                out_hbm.at[base],
                wsem.at[b],
            ).wait()

    return gather_kernel(table, ipad)


# speedup vs baseline: 1.2787x; 1.2787x over previous
import jax
import jax.numpy as jnp
from jax import lax
from jax.experimental import pallas as pl
from jax.experimental.pallas import tpu as pltpu
from jax.experimental.pallas import tpu_sc as plsc

_NC = 2
_NS = 16
_NW = _NC * _NS
_C = 128
_NBUF = 8


def kernel(segment_ids, table):
    batch, hist = segment_ids.shape
    num_rows, dim = table.shape
    total = batch * hist
    per_w = total // _NW
    n_chunks = per_w // _C
    assert total % _NW == 0 and per_w % _C == 0 and n_chunks % _NBUF == 0

    flat_idx = segment_ids.reshape(total).astype(jnp.int32)
    mesh = plsc.VectorSubcoreMesh(core_axis_name="c", subcore_axis_name="s")

    @pl.kernel(
        out_type=jax.ShapeDtypeStruct((total, dim), table.dtype),
        mesh=mesh,
        scratch_types=[
            pltpu.VMEM((per_w,), jnp.int32),
            pltpu.VMEM((_NBUF, _C, dim), jnp.float32),
            pltpu.SemaphoreType.DMA((_NBUF,)),
            pltpu.SemaphoreType.DMA((_NBUF,)),
        ],
        compiler_params=pltpu.CompilerParams(use_tc_tiling_on_sc=False),
    )
    def gather_kernel(table_hbm, idx_hbm, out_hbm, idx_v, rows_v, gsem, wsem):
        wid = lax.axis_index("s") * _NC + lax.axis_index("c")
        base = wid * per_w
        pltpu.sync_copy(idx_hbm.at[pl.ds(base, per_w)], idx_v)

        def fire_gather(g, b):
            pltpu.async_copy(
                table_hbm.at[idx_v.at[pl.ds(g * _C, _C)]],
                rows_v.at[b],
                gsem.at[b],
            )

        for b in range(_NBUF):
            fire_gather(b, b)

        @pl.loop(0, n_chunks, step=_NBUF)
        def _(g0):
            for b in range(_NBUF):
                g = g0 + b
                pltpu.make_async_copy(
                    table_hbm.at[idx_v.at[pl.ds(g * _C, _C)]],
                    rows_v.at[b],
                    gsem.at[b],
                ).wait()
                wb = pltpu.async_copy(
                    rows_v.at[b],
                    out_hbm.at[pl.ds(base + g * _C, _C)],
                    wsem.at[b],
                )

                @pl.when(g + _NBUF < n_chunks)
                def _():
                    wb.wait()
                    fire_gather(g + _NBUF, b)

        for b in range(_NBUF):
            pltpu.make_async_copy(
                rows_v.at[b],
                out_hbm.at[pl.ds(base, _C)],
                wsem.at[b],
            ).wait()

    out = gather_kernel(table, flat_idx)
    return out.reshape(batch, hist, dim)
